# baseline (speedup 1.0000x reference)
import functools

import jax
import jax.numpy as jnp
from jax.experimental import pallas as pl

D = 128
BU_BLK = 1024


def _gru_ln_body(x_ref, h_ref, wih_ref, whh_ref, bih_ref, bhh_ref, g_ref, be_ref, o_ref):
    x = x_ref[...]
    h = h_ref[...]
    gi = jax.lax.dot_general(x, wih_ref[...], dimension_numbers=(((1,), (1,)), ((), ())),
                             preferred_element_type=jnp.float32) + bih_ref[...]
    gh = jax.lax.dot_general(h, whh_ref[...], dimension_numbers=(((1,), (1,)), ((), ())),
                             preferred_element_type=jnp.float32) + bhh_ref[...]
    i_r, i_z, i_n = gi[:, :D], gi[:, D:2 * D], gi[:, 2 * D:]
    h_r, h_z, h_n = gh[:, :D], gh[:, D:2 * D], gh[:, 2 * D:]
    r = jax.nn.sigmoid(i_r + h_r)
    z = jax.nn.sigmoid(i_z + h_z)
    n = jnp.tanh(i_n + r * h_n)
    hn = (1.0 - z) * n + z * h
    mu = jnp.mean(hn, axis=-1, keepdims=True)
    var = jnp.mean((hn - mu) ** 2, axis=-1, keepdims=True)
    o_ref[...] = (hn - mu) / jnp.sqrt(var + 1e-5) * g_ref[...] + be_ref[...]


@functools.partial(jax.jit, static_argnames=())
def _gru_ln(x, h, W_ih, W_hh, b_ih, b_hh, gamma, beta):
    bu = x.shape[0]
    grid = (bu // BU_BLK,)
    return pl.pallas_call(
        _gru_ln_body,
        grid=grid,
        in_specs=[
            pl.BlockSpec((BU_BLK, D), lambda i: (i, 0)),
            pl.BlockSpec((BU_BLK, D), lambda i: (i, 0)),
            pl.BlockSpec((3 * D, D), lambda i: (0, 0)),
            pl.BlockSpec((3 * D, D), lambda i: (0, 0)),
            pl.BlockSpec((1, 3 * D), lambda i: (0, 0)),
            pl.BlockSpec((1, 3 * D), lambda i: (0, 0)),
            pl.BlockSpec((1, D), lambda i: (0, 0)),
            pl.BlockSpec((1, D), lambda i: (0, 0)),
        ],
        out_specs=pl.BlockSpec((BU_BLK, D), lambda i: (i, 0)),
        out_shape=jax.ShapeDtypeStruct((bu, D), jnp.float32),
    )(x, h, W_ih, W_hh, b_ih, b_hh, gamma, beta)


def kernel(mem, last_update, unique_messages, timestamps, W_ih, W_hh, b_ih, b_hh,
           gamma, beta, node_ids, to_update_node_ids, node_type):
    B = node_ids.shape[0]
    BU = to_update_node_ids.shape[0]
    tud = to_update_node_ids.astype(jnp.int32)
    nid = node_ids.astype(jnp.int32)

    gidx = jnp.take(nid, tud, axis=0)
    h = jnp.take(mem, gidx, axis=0)
    h_new = _gru_ln(unique_messages, h,
                    W_ih, W_hh, b_ih.reshape(1, -1), b_hh.reshape(1, -1),
                    gamma.reshape(1, -1), beta.reshape(1, -1))

    winner = jnp.full((B,), -1, dtype=jnp.int32).at[tud].max(
        jnp.arange(BU, dtype=jnp.int32), mode="drop")
    win_clip = jnp.maximum(winner, 0)
    has = winner >= 0

    gathered = jnp.take(mem, nid, axis=0)
    sel = jnp.take(h_new, win_clip, axis=0)
    updated_memory = jnp.where(has[:, None], sel, gathered)

    lu_g = jnp.take(last_update, nid, axis=0)
    ts_sel = jnp.take(timestamps, win_clip, axis=0)
    updated_last_update = jnp.where(has, ts_sel, lu_g)
    return (updated_memory, updated_last_update)
